# baseline (speedup 1.0000x reference)
import jax
import jax.numpy as jnp
from jax import lax
from jax.experimental import pallas as pl
from jax.experimental.pallas import tpu as pltpu
from jax.experimental.pallas import tpu_sc as plsc

N = 10000
NP = 10240
E = 160000
H = 256
L = 5
G = 16
H3 = 3 * H
HH = H // 2
NSUB = 16
C = 128
NCH = 80
E2 = NSUB * NCH * C
RPT = NP // NSUB
BLK = 1024

_PREC = lax.Precision.HIGHEST


def _sc_body(hlo, hhi, src3, dst3, attr3, zrow,
             agg_lo, agg_hi,
             src_v, dst_v, attr_v, rows, acc, sem):
    cid = lax.axis_index("c")
    sid = lax.axis_index("s")

    pltpu.sync_copy(src3.at[sid], src_v)
    pltpu.sync_copy(dst3.at[sid], dst_v)
    pltpu.sync_copy(attr3.at[sid], attr_v)

    stripe = pl.ds(sid * RPT, RPT)
    pltpu.sync_copy(zrow, acc.at[stripe])
    plsc.subcore_barrier()

    def chunk(j, carry):
        idx = src_v.at[j]

        @pl.when(cid == 0)
        def _():
            pltpu.async_copy(hlo.at[idx], rows, sem).wait()

        @pl.when(cid == 1)
        def _():
            pltpu.async_copy(hhi.at[idx], rows, sem).wait()

        for g in range(C // 16):
            av = attr_v[j, pl.ds(g * 16, 16)]
            for t in range(16):
                a = av[t]
                row = g * 16 + t
                for k in range(HH // 16):
                    sl = pl.ds(k * 16, 16)
                    rows[row, sl] = rows[row, sl] * a

        pltpu.sync_copy(rows, acc.at[dst_v.at[j]], add=True)
        return carry

    lax.fori_loop(0, NCH, chunk, 0)
    plsc.subcore_barrier()

    @pl.when(cid == 0)
    def _():
        pltpu.sync_copy(acc.at[stripe], agg_lo.at[stripe])

    @pl.when(cid == 1)
    def _():
        pltpu.sync_copy(acc.at[stripe], agg_hi.at[stripe])


def _sc_agg(hlo, hhi, src3, dst3, attr3, zrow):
    f = pl.kernel(
        _sc_body,
        out_type=[jax.ShapeDtypeStruct((NP, HH), jnp.float32),
                  jax.ShapeDtypeStruct((NP, HH), jnp.float32)],
        mesh=plsc.VectorSubcoreMesh(core_axis_name="c", subcore_axis_name="s"),
        scratch_types=[
            pltpu.VMEM((NCH, C), jnp.int32),
            pltpu.VMEM((NCH, C), jnp.int32),
            pltpu.VMEM((NCH, C), jnp.float32),
            pltpu.VMEM((C, HH), jnp.float32),
            pltpu.VMEM_SHARED((NP, HH), jnp.float32),
            pltpu.SemaphoreType.DMA,
        ],
    )
    return f(hlo, hhi, src3, dst3, attr3, zrow)


def _bmm_body(w_ref, wih_ref, b_ref):
    b_ref[0] = jnp.dot(w_ref[0], wih_ref[...],
                       preferred_element_type=jnp.float32, precision=_PREC)


def _bmm(weight, wih_t):
    return pl.pallas_call(
        _bmm_body,
        grid=(L,),
        in_specs=[pl.BlockSpec((1, H, H), lambda i: (i, 0, 0)),
                  pl.BlockSpec((H, H3), lambda i: (0, 0))],
        out_specs=pl.BlockSpec((1, H, H3), lambda i: (i, 0, 0)),
        out_shape=jax.ShapeDtypeStruct((L, H, H3), jnp.float32),
    )(weight, wih_t)


def _gru(hlo, hhi, alo, ahi, b_l, whh_t, bih, bhh):
    h = jnp.concatenate([hlo, hhi], axis=1)
    hagg = jnp.concatenate([alo, ahi], axis=1)
    gi = jnp.dot(hagg, b_l, preferred_element_type=jnp.float32,
                 precision=_PREC) + bih
    gh = jnp.dot(h, whh_t, preferred_element_type=jnp.float32,
                 precision=_PREC) + bhh
    r = jax.nn.sigmoid(gi[:, :H] + gh[:, :H])
    z = jax.nn.sigmoid(gi[:, H:2 * H] + gh[:, H:2 * H])
    n = jnp.tanh(gi[:, 2 * H:] + r * gh[:, 2 * H:])
    return (1.0 - z) * n + z * h


def _gru_body(hlo_ref, hhi_ref, alo_ref, ahi_ref, b_ref, whh_ref,
              bih_ref, bhh_ref, nlo_ref, nhi_ref):
    hn = _gru(hlo_ref[...], hhi_ref[...], alo_ref[...], ahi_ref[...],
              b_ref[...], whh_ref[...], bih_ref[...], bhh_ref[...])
    nlo_ref[...] = hn[:, :HH]
    nhi_ref[...] = hn[:, HH:]


def _gru_step(hlo, hhi, alo, ahi, b_l, whh_t, bih, bhh):
    full = lambda r, c: pl.BlockSpec((r, c), lambda i: (0, 0))
    row = lambda c: pl.BlockSpec((BLK, c), lambda i: (i, 0))
    return pl.pallas_call(
        _gru_body,
        grid=(NP // BLK,),
        in_specs=[row(HH), row(HH), row(HH), row(HH), full(H, H3),
                  full(H, H3), full(1, H3), full(1, H3)],
        out_specs=[row(HH), row(HH)],
        out_shape=[jax.ShapeDtypeStruct((NP, HH), jnp.float32)] * 2,
    )(hlo, hhi, alo, ahi, b_l, whh_t, bih, bhh)


def _final_body(hlo_ref, hhi_ref, alo_ref, ahi_ref, b_ref, whh_ref,
                bih_ref, bhh_ref, fcw_ref, fcb_ref, bt_ref, out_ref, acc_ref):
    i = pl.program_id(0)
    hn = _gru(hlo_ref[...], hhi_ref[...], alo_ref[...], ahi_ref[...],
              b_ref[...], whh_ref[...], bih_ref[...], bhh_ref[...])
    hr = jax.nn.relu(hn)
    fcm = jnp.sum(fcw_ref[...], axis=0, keepdims=True) * 0.01
    bm = jnp.sum(fcb_ref[...]) * 0.01
    v = jnp.sum(hr * fcm, axis=1, keepdims=True) + bm
    onehot = (bt_ref[...] ==
              lax.broadcasted_iota(jnp.int32, (1, 128), 1)).astype(jnp.float32)
    sums = jnp.sum(v * onehot, axis=0, keepdims=True)
    cnts = jnp.sum(onehot, axis=0, keepdims=True)

    @pl.when(i == 0)
    def _():
        acc_ref[...] = jnp.zeros_like(acc_ref)

    acc_ref[0:1, :] += sums
    acc_ref[1:2, :] += cnts

    @pl.when(i == pl.num_programs(0) - 1)
    def _():
        out_ref[...] = acc_ref[0:1, :] / jnp.maximum(acc_ref[1:2, :], 1.0)


def _final_step(hlo, hhi, alo, ahi, b_l, whh_t, bih, bhh, fcw_p, fcb_p, bt):
    full = lambda r, c: pl.BlockSpec((r, c), lambda i: (0, 0))
    row = lambda c: pl.BlockSpec((BLK, c), lambda i: (i, 0))
    return pl.pallas_call(
        _final_body,
        grid=(NP // BLK,),
        in_specs=[row(HH), row(HH), row(HH), row(HH), full(H, H3),
                  full(H, H3), full(1, H3), full(1, H3), full(104, H),
                  full(1, 128), row(1)],
        out_specs=pl.BlockSpec((1, 128), lambda i: (0, 0)),
        out_shape=jax.ShapeDtypeStruct((1, 128), jnp.float32),
        scratch_shapes=[pltpu.VMEM((8, 128), jnp.float32)],
    )(hlo, hhi, alo, ahi, b_l, whh_t, bih, bhh, fcw_p, fcb_p, bt)


def kernel(x, edge_index, edge_attr, mask, batch, weight, w_ih, w_hh,
           b_ih, b_hh, fc_w, fc_b):
    pad = E2 - E
    src3 = jnp.pad(edge_index[0].astype(jnp.int32),
                   (0, pad)).reshape(NSUB, NCH, C)
    dst3 = jnp.pad(edge_index[1].astype(jnp.int32), (0, pad),
                   constant_values=0)
    dst3 = dst3.at[E:].set(N + jnp.arange(pad, dtype=jnp.int32) % (NP - N))
    dst3 = dst3.reshape(NSUB, NCH, C)
    attr3 = jnp.pad(edge_attr, (0, pad)).reshape(NSUB, NCH, C)
    zrow = jnp.zeros((RPT, HH), jnp.float32)
    bih = b_ih.reshape(1, H3)
    bhh = b_hh.reshape(1, H3)
    fcw_p = jnp.pad(fc_w, ((0, 4), (0, 0)))
    fcb_p = jnp.pad(fc_b, (0, 28)).reshape(1, 128)
    bt = jnp.pad(batch.astype(jnp.int32), (0, NP - N),
                 constant_values=G).reshape(NP, 1)

    bb = _bmm(weight, w_ih.T)
    hp = jnp.pad(x, ((0, NP - N), (0, 0)))
    hlo, hhi = hp[:, :HH], hp[:, HH:]
    for l in range(L):
        alo, ahi = _sc_agg(hlo, hhi, src3, dst3, attr3, zrow)
        if l < L - 1:
            hlo, hhi = _gru_step(hlo, hhi, alo, ahi, bb[l], w_hh.T, bih, bhh)
        else:
            out = _final_step(hlo, hhi, alo, ahi, bb[l], w_hh.T, bih, bhh,
                              fcw_p, fcb_p, bt)
    return out[0, :G]


# speedup vs baseline: 1.1925x; 1.1925x over previous
import jax
import jax.numpy as jnp
from jax import lax
from jax.experimental import pallas as pl
from jax.experimental.pallas import tpu as pltpu
from jax.experimental.pallas import tpu_sc as plsc

N = 10000
NP = 10240
E = 160000
H = 256
L = 5
G = 16
H3 = 3 * H
HH = H // 2
NSUB = 16
C = 128
NCH = -(-E // (NSUB * C))
E2 = NSUB * NCH * C
RPT = NP // NSUB
BLK = 1024

_PREC = lax.Precision.HIGHEST


def _sc_body(mlo, mhi, src3, dst3, attr3, zrow,
             agg_lo, agg_hi,
             src_v, dst_v, attr_v, rows, acc, sem):
    cid = lax.axis_index("c")
    sid = lax.axis_index("s")

    pltpu.sync_copy(src3.at[sid], src_v)
    pltpu.sync_copy(dst3.at[sid], dst_v)
    pltpu.sync_copy(attr3.at[sid], attr_v)

    stripe = pl.ds(sid * RPT, RPT)
    pltpu.sync_copy(zrow, acc.at[stripe])
    plsc.subcore_barrier()

    def chunk(j, carry):
        idx = src_v.at[j]

        @pl.when(cid == 0)
        def _():
            pltpu.async_copy(mlo.at[idx], rows, sem).wait()

        @pl.when(cid == 1)
        def _():
            pltpu.async_copy(mhi.at[idx], rows, sem).wait()

        for g in range(C // 16):
            av = attr_v[j, pl.ds(g * 16, 16)]
            for t in range(16):
                a = av[t]
                e = g * 16 + t
                for k in range(HH // 16):
                    sl = pl.ds(k * 16, 16)
                    rows[e, sl] = rows[e, sl] * a

        pltpu.sync_copy(rows, acc.at[dst_v.at[j]], add=True)
        return carry

    lax.fori_loop(0, NCH, chunk, 0)
    plsc.subcore_barrier()

    @pl.when(cid == 0)
    def _():
        pltpu.sync_copy(acc.at[stripe], agg_lo.at[stripe])

    @pl.when(cid == 1)
    def _():
        pltpu.sync_copy(acc.at[stripe], agg_hi.at[stripe])


def _sc_agg(mlo, mhi, src3, dst3, attr3, zrow):
    f = pl.kernel(
        _sc_body,
        out_type=[jax.ShapeDtypeStruct((NP, HH), jnp.float32),
                  jax.ShapeDtypeStruct((NP, HH), jnp.float32)],
        mesh=plsc.VectorSubcoreMesh(core_axis_name="c", subcore_axis_name="s"),
        scratch_types=[
            pltpu.VMEM((NCH, C), jnp.int32),
            pltpu.VMEM((NCH, C), jnp.int32),
            pltpu.VMEM((NCH, C), jnp.float32),
            pltpu.VMEM((C, HH), jnp.float32),
            pltpu.VMEM_SHARED((NP, HH), jnp.float32),
            pltpu.SemaphoreType.DMA,
        ],
    )
    return f(mlo, mhi, src3, dst3, attr3, zrow)


def _mm_body(x_ref, w_ref, lo_ref, hi_ref):
    m = jnp.dot(x_ref[...], w_ref[...],
                preferred_element_type=jnp.float32, precision=_PREC)
    lo_ref[...] = m[:, :HH]
    hi_ref[...] = m[:, HH:]


def _first_matmul(x, w):
    return pl.pallas_call(
        _mm_body,
        grid=(NP // BLK,),
        in_specs=[pl.BlockSpec((BLK, H), lambda i: (i, 0)),
                  pl.BlockSpec((H, H), lambda i: (0, 0))],
        out_specs=[pl.BlockSpec((BLK, HH), lambda i: (i, 0)),
                   pl.BlockSpec((BLK, HH), lambda i: (i, 0))],
        out_shape=[jax.ShapeDtypeStruct((NP, HH), jnp.float32)] * 2,
    )(x, w)


def _gru(h, alo, ahi, wih_t, whh_t, bih, bhh):
    agg = jnp.concatenate([alo, ahi], axis=1)
    gi = jnp.dot(agg, wih_t, preferred_element_type=jnp.float32,
                 precision=_PREC) + bih
    gh = jnp.dot(h, whh_t, preferred_element_type=jnp.float32,
                 precision=_PREC) + bhh
    r = jax.nn.sigmoid(gi[:, :H] + gh[:, :H])
    z = jax.nn.sigmoid(gi[:, H:2 * H] + gh[:, H:2 * H])
    n = jnp.tanh(gi[:, 2 * H:] + r * gh[:, 2 * H:])
    return (1.0 - z) * n + z * h


def _gru_body(h_ref, alo_ref, ahi_ref, wih_ref, whh_ref, bih_ref, bhh_ref,
              wn_ref, hn_ref, lo_ref, hi_ref):
    hn = _gru(h_ref[...], alo_ref[...], ahi_ref[...], wih_ref[...],
              whh_ref[...], bih_ref[...], bhh_ref[...])
    hn_ref[...] = hn
    m = jnp.dot(hn, wn_ref[...], preferred_element_type=jnp.float32,
                precision=_PREC)
    lo_ref[...] = m[:, :HH]
    hi_ref[...] = m[:, HH:]


def _gru_step(h, alo, ahi, wih_t, whh_t, bih, bhh, w_next):
    full = lambda r, c: pl.BlockSpec((r, c), lambda i: (0, 0))
    row = lambda c: pl.BlockSpec((BLK, c), lambda i: (i, 0))
    return pl.pallas_call(
        _gru_body,
        grid=(NP // BLK,),
        in_specs=[row(H), row(HH), row(HH), full(H, H3), full(H, H3),
                  full(1, H3), full(1, H3), full(H, H)],
        out_specs=[row(H), row(HH), row(HH)],
        out_shape=[jax.ShapeDtypeStruct((NP, H), jnp.float32),
                   jax.ShapeDtypeStruct((NP, HH), jnp.float32),
                   jax.ShapeDtypeStruct((NP, HH), jnp.float32)],
    )(h, alo, ahi, wih_t, whh_t, bih, bhh, w_next)


def _final_body(h_ref, alo_ref, ahi_ref, wih_ref, whh_ref, bih_ref, bhh_ref,
                fcw_ref, fcb_ref, bt_ref, out_ref, acc_ref):
    i = pl.program_id(0)
    hn = _gru(h_ref[...], alo_ref[...], ahi_ref[...], wih_ref[...],
              whh_ref[...], bih_ref[...], bhh_ref[...])
    hr = jax.nn.relu(hn)
    fcm = jnp.sum(fcw_ref[...], axis=0, keepdims=True) * 0.01
    bm = jnp.sum(fcb_ref[...]) * 0.01
    v = jnp.sum(hr * fcm, axis=1, keepdims=True) + bm
    onehot = (bt_ref[...] ==
              lax.broadcasted_iota(jnp.int32, (1, 128), 1)).astype(jnp.float32)
    sums = jnp.sum(v * onehot, axis=0, keepdims=True)
    cnts = jnp.sum(onehot, axis=0, keepdims=True)

    @pl.when(i == 0)
    def _():
        acc_ref[...] = jnp.zeros_like(acc_ref)

    acc_ref[0:1, :] += sums
    acc_ref[1:2, :] += cnts

    @pl.when(i == pl.num_programs(0) - 1)
    def _():
        out_ref[...] = acc_ref[0:1, :] / jnp.maximum(acc_ref[1:2, :], 1.0)


def _final_step(h, alo, ahi, wih_t, whh_t, bih, bhh, fcw_p, fcb_p, bt):
    full = lambda r, c: pl.BlockSpec((r, c), lambda i: (0, 0))
    row = lambda c: pl.BlockSpec((BLK, c), lambda i: (i, 0))
    return pl.pallas_call(
        _final_body,
        grid=(NP // BLK,),
        in_specs=[row(H), row(HH), row(HH), full(H, H3), full(H, H3),
                  full(1, H3), full(1, H3), full(104, H), full(1, 128),
                  row(1)],
        out_specs=pl.BlockSpec((1, 128), lambda i: (0, 0)),
        out_shape=jax.ShapeDtypeStruct((1, 128), jnp.float32),
        scratch_shapes=[pltpu.VMEM((8, 128), jnp.float32)],
    )(h, alo, ahi, wih_t, whh_t, bih, bhh, fcw_p, fcb_p, bt)


def kernel(x, edge_index, edge_attr, mask, batch, weight, w_ih, w_hh,
           b_ih, b_hh, fc_w, fc_b):
    pad = E2 - E
    src3 = jnp.pad(edge_index[0].astype(jnp.int32),
                   (0, pad)).reshape(NSUB, NCH, C)
    dst3 = jnp.pad(edge_index[1].astype(jnp.int32), (0, pad),
                   constant_values=0)
    dst3 = dst3.at[E:].set(N + jnp.arange(pad, dtype=jnp.int32) % (NP - N))
    dst3 = dst3.reshape(NSUB, NCH, C)
    attr3 = jnp.pad(edge_attr, (0, pad)).reshape(NSUB, NCH, C)
    zrow = jnp.zeros((RPT, HH), jnp.float32)
    wih_t = w_ih.T
    whh_t = w_hh.T
    bih = b_ih.reshape(1, H3)
    bhh = b_hh.reshape(1, H3)
    fcw_p = jnp.pad(fc_w, ((0, 4), (0, 0)))
    fcb_p = jnp.pad(fc_b, (0, 28)).reshape(1, 128)
    bt = jnp.pad(batch.astype(jnp.int32), (0, NP - N),
                 constant_values=G).reshape(NP, 1)

    h = jnp.pad(x, ((0, NP - N), (0, 0)))
    mlo, mhi = _first_matmul(h, weight[0])
    for l in range(L):
        alo, ahi = _sc_agg(mlo, mhi, src3, dst3, attr3, zrow)
        if l < L - 1:
            h, mlo, mhi = _gru_step(h, alo, ahi, wih_t, whh_t, bih, bhh,
                                    weight[l + 1])
        else:
            out = _final_step(h, alo, ahi, wih_t, whh_t, bih, bhh,
                              fcw_p, fcb_p, bt)
    return out[0, :G]
